# baseline (speedup 1.0000x reference)
import functools

import jax
import jax.numpy as jnp
from jax import lax
from jax.experimental import pallas as pl
from jax.experimental.pallas import tpu as pltpu
from jax.experimental.pallas import tpu_sc as plsc

_COUNT = 10
_BUCKET_SIZE = 3
_LANES = 16
_NC, _NS = 2, 16
_NW = _NC * _NS


def _make_sc_kernel(batch, d):
    chunk = batch // _NW
    groups = chunk // _LANES
    mesh = plsc.VectorSubcoreMesh(
        core_axis_name="c", subcore_axis_name="s",
        num_cores=_NC, num_subcores=_NS,
    )

    @functools.partial(
        pl.kernel,
        mesh=mesh,
        compiler_params=pltpu.CompilerParams(needs_layout_passes=False),
        out_type=jax.ShapeDtypeStruct((batch,), jnp.float32),
        scratch_types=[
            pltpu.VMEM((chunk * d,), jnp.float32),
            pltpu.VMEM((_COUNT * d,), jnp.float32),
            pltpu.VMEM((_COUNT,), jnp.float32),
            pltpu.VMEM((chunk,), jnp.int32),
            pltpu.VMEM((chunk,), jnp.float32),
        ],
    )
    def _sc_kernel(x_hbm, ply_hbm, w_hbm, b_hbm, out_hbm, x_v, w_v, b_v, ply_v, res_v):
        wid = lax.axis_index("s") * _NC + lax.axis_index("c")
        base = wid * chunk
        pltpu.sync_copy(x_hbm.at[pl.ds(base * d, chunk * d)], x_v)
        pltpu.sync_copy(w_hbm, w_v)
        pltpu.sync_copy(b_hbm, b_v)
        pltpu.sync_copy(ply_hbm.at[pl.ds(base, chunk)], ply_v)

        lane = lax.iota(jnp.int32, _LANES)
        three = jnp.full((_LANES,), _BUCKET_SIZE, jnp.int32)
        dsplat = jnp.full((_LANES,), d, jnp.int32)
        lane_d = lax.mul(lane, dsplat)

        def body(g, carry):
            c = lax.div(ply_v[pl.ds(lax.mul(g, _LANES), _LANES)], three)
            wb = lax.mul(c, dsplat)
            g_off = lax.broadcast(lax.mul(g, _LANES * d), (_LANES,))
            rb = lax.add(lane_d, g_off)
            acc = jnp.zeros((_LANES,), jnp.float32)
            for k in range(d):
                ks = jnp.full((_LANES,), k, jnp.int32)
                xv = plsc.load_gather(x_v, [lax.add(rb, ks)])
                wv = plsc.load_gather(w_v, [lax.add(wb, ks)])
                acc = lax.add(acc, lax.mul(xv, wv))
            bv = plsc.load_gather(b_v, [c])
            res_v[pl.ds(lax.mul(g, _LANES), _LANES)] = lax.add(acc, bv)
            return carry

        lax.fori_loop(0, groups, body, 0)
        pltpu.sync_copy(res_v, out_hbm.at[pl.ds(base, chunk)])

    return _sc_kernel


def kernel(x, ply, W, b):
    batch, d = x.shape
    out = _make_sc_kernel(batch, d)(x.reshape(-1), ply, W.reshape(-1), b)
    return out.reshape(batch, 1)


# speedup vs baseline: 14.8116x; 14.8116x over previous
import jax
import jax.numpy as jnp
from jax import lax
from jax.experimental import pallas as pl

_COUNT = 10
_BUCKET_SIZE = 3


def _fused_body(x_ref, ply_ref, w_ref, b_ref, o_ref):
    full_t = lax.dot_general(
        w_ref[...], x_ref[...],
        dimension_numbers=(((1,), (1,)), ((), ())),
        preferred_element_type=jnp.float32,
        precision=lax.Precision.DEFAULT,
    )
    c = ply_ref[0] // _BUCKET_SIZE
    rows = lax.broadcasted_iota(jnp.int32, (_COUNT, 1), 0)
    mask = c == rows
    sel = jnp.sum(jnp.where(mask, full_t + b_ref[...], 0.0), axis=0, keepdims=True)
    o_ref[0] = sel


def kernel(x, ply, W, b):
    batch, d = x.shape
    blk = 8192
    nblk = batch // blk
    out = pl.pallas_call(
        _fused_body,
        grid=(nblk,),
        in_specs=[
            pl.BlockSpec((blk, d), lambda i: (i, 0)),
            pl.BlockSpec((1, 1, blk), lambda i: (i, 0, 0)),
            pl.BlockSpec((_COUNT, d), lambda i: (0, 0)),
            pl.BlockSpec((_COUNT, 1), lambda i: (0, 0)),
        ],
        out_specs=pl.BlockSpec((1, 1, blk), lambda i: (i, 0, 0)),
        out_shape=jax.ShapeDtypeStruct((nblk, 1, blk), jnp.float32),
    )(x, ply.reshape(nblk, 1, blk), W, b.reshape(_COUNT, 1))
    return out.reshape(batch, 1)
